# Initial kernel scaffold; baseline (speedup 1.0000x reference)
import jax
import jax.numpy as jnp
from jax.experimental import pallas as pl


def kernel(h, diff_cart, dist_sq, edge_src, edge_dst, t_emb_edges, t_emb_nodes, W_e1, b_e1, W_e2, b_e2, W_c1, b_c1, W_c2, W_n1, b_n1, W_n2, b_n2):
    raise NotImplementedError("write your pallas kernel here")


# speedup vs baseline: 1.8326x; 1.8326x over previous
import functools

import jax
import jax.numpy as jnp
from jax import lax
from jax.experimental import pallas as pl
from jax.experimental.pallas import tpu as pltpu
from jax.experimental.pallas import tpu_sc as plsc

_SC_PARAMS = dict(
    compiler_params=pltpu.CompilerParams(needs_layout_passes=False),
)

_CHI = 128


def _sc_mesh():
    return plsc.VectorSubcoreMesh(core_axis_name="c", subcore_axis_name="s")


def _row_partition(M, NS):
    rpt = -(-(M // NS) // 8) * 8
    rlast = M - rpt * (NS - 1)
    assert rlast > 0 and rpt % 8 == 0 and rlast % 8 == 0, (M, NS)
    return rpt, rlast


def _chunking(ept):
    nch = ept // _CHI
    tail = ept - nch * _CHI
    assert tail % 8 == 0
    return nch, tail


def _stage_idx(idx, nways, nch, tail):
    ept = nch * _CHI + tail
    i2 = idx.reshape(nways, ept)
    main = i2[:, :nch * _CHI].reshape(nways, nch, _CHI)
    tl = i2[:, nch * _CHI:]
    return main, tl


def _make_sc_gather(N, D, E):
    info = plsc.get_sparse_core_info()
    NC, NS = info.num_cores, info.num_subcores
    NW = NC * NS
    assert E % NW == 0
    EPT = E // NW
    NCH, TAIL = _chunking(EPT)

    @functools.partial(
        pl.kernel, mesh=_sc_mesh(), **_SC_PARAMS,
        out_type=[jax.ShapeDtypeStruct((E, D), jnp.float32),
                  jax.ShapeDtypeStruct((E, D), jnp.float32)],
        scratch_types=[
            pltpu.VMEM((_CHI,), jnp.int32),
            pltpu.VMEM((_CHI,), jnp.int32),
            pltpu.VMEM((TAIL,), jnp.int32),
            pltpu.VMEM((TAIL,), jnp.int32),
            pltpu.VMEM((_CHI, D), jnp.float32),
            pltpu.VMEM((_CHI, D), jnp.float32),
            pltpu.VMEM((TAIL, D), jnp.float32),
            pltpu.SemaphoreType.DMA,
            pltpu.SemaphoreType.DMA,
        ],
    )
    def gather_k(h_hbm, src2_hbm, srct_hbm, dst2_hbm, dstt_hbm,
                 hs_out, hd_out,
                 row_s, row_d, idxt_s, idxt_d, buf_a, buf_b, buf_t,
                 sem_a, sem_b):
        wid = lax.axis_index("s") * NC + lax.axis_index("c")
        base = wid * EPT
        pltpu.sync_copy(srct_hbm.at[wid], idxt_s)
        pltpu.sync_copy(dstt_hbm.at[wid], idxt_d)

        def body(c, carry):
            off = base + c * _CHI
            pltpu.sync_copy(src2_hbm.at[wid, c], row_s)
            pltpu.sync_copy(dst2_hbm.at[wid, c], row_d)
            ca = pltpu.async_copy(h_hbm.at[row_s], buf_a, sem_a)
            cb = pltpu.async_copy(h_hbm.at[row_d], buf_b, sem_b)
            ca.wait()
            pltpu.sync_copy(buf_a, hs_out.at[pl.ds(off, _CHI)])
            cb.wait()
            pltpu.sync_copy(buf_b, hd_out.at[pl.ds(off, _CHI)])
            return carry

        lax.fori_loop(0, NCH, body, 0)

        toff = base + NCH * _CHI
        pltpu.async_copy(h_hbm.at[idxt_s], buf_t, sem_a).wait()
        pltpu.sync_copy(buf_t, hs_out.at[pl.ds(toff, TAIL)])
        pltpu.async_copy(h_hbm.at[idxt_d], buf_t, sem_a).wait()
        pltpu.sync_copy(buf_t, hd_out.at[pl.ds(toff, TAIL)])

    return gather_k


def _make_sc_scatter(N, D, E):
    info = plsc.get_sparse_core_info()
    NC, NS = info.num_cores, info.num_subcores
    NW = NC * NS
    EPC = E // NS
    NCHM, TAILM = _chunking(EPC)
    EPT = E // NW
    NCHC, TAILC = _chunking(EPT)
    HN = N // NC
    HACC = -(-(HN + 1) // 8) * 8
    RPT, RLAST = _row_partition(HACC, NS)
    CLEN = N * 4

    @functools.partial(
        pl.kernel, mesh=_sc_mesh(), **_SC_PARAMS,
        out_type=[jax.ShapeDtypeStruct((NC, HACC, D), jnp.float32),
                  jax.ShapeDtypeStruct((NW, CLEN), jnp.float32)],
        scratch_types=[
            pltpu.VMEM((_CHI,), jnp.int32),
            pltpu.VMEM((_CHI,), jnp.int32),
            pltpu.VMEM((TAILM,), jnp.int32),
            pltpu.VMEM((TAILM,), jnp.int32),
            pltpu.VMEM((_CHI, D), jnp.float32),
            pltpu.VMEM((TAILM, D), jnp.float32),
            pltpu.VMEM((_CHI,), jnp.int32),
            pltpu.VMEM((_CHI,), jnp.float32),
            pltpu.VMEM((3 * _CHI,), jnp.float32),
            pltpu.VMEM((CLEN,), jnp.float32),
            pltpu.VMEM_SHARED((HACC, D), jnp.float32),
        ],
    )
    def scatter_k(m_hbm, dst2_hbm, dstt_hbm, src_hbm, cw_hbm, diff_hbm,
                  zm_hbm, zc_hbm,
                  mp_out, cp_out,
                  row_d, idx_buf, rowt_d, idxt_buf, m_buf, m_buft,
                  srci, cwb, diffb, cacc, sp_m):
        cid = lax.axis_index("c")
        sid = lax.axis_index("s")
        wid = sid * NC + cid
        lo = cid * HN
        iota16 = lax.iota(jnp.int32, 16)

        @pl.when(sid < NS - 1)
        def _():
            pltpu.sync_copy(zm_hbm.at[pl.ds(sid * RPT, RPT)],
                            sp_m.at[pl.ds(sid * RPT, RPT)])

        @pl.when(sid == NS - 1)
        def _():
            pltpu.sync_copy(zm_hbm.at[pl.ds((NS - 1) * RPT, RLAST)],
                            sp_m.at[pl.ds((NS - 1) * RPT, RLAST)])

        pltpu.sync_copy(zc_hbm, cacc)
        plsc.subcore_barrier()

        base_e = sid * EPC

        def remap(n16, raw_ref, out_ref):
            def rbody(j, carry):
                v = raw_ref[pl.ds(j * 16, 16)]
                t = v - lo
                ok = (t >= 0) & (t < HN)
                out_ref[pl.ds(j * 16, 16)] = jnp.where(ok, t, HN)
                return carry
            lax.fori_loop(0, n16 // 16, rbody, 0)

        def mbody(c, carry):
            off = base_e + c * _CHI
            pltpu.sync_copy(dst2_hbm.at[sid, c], row_d)
            pltpu.sync_copy(m_hbm.at[pl.ds(off, _CHI)], m_buf)
            remap(_CHI, row_d, idx_buf)
            pltpu.sync_copy(m_buf, sp_m.at[idx_buf], add=True)
            return carry

        lax.fori_loop(0, NCHM, mbody, 0)

        toff = base_e + NCHM * _CHI
        pltpu.sync_copy(dstt_hbm.at[sid], rowt_d)
        pltpu.sync_copy(m_hbm.at[pl.ds(toff, TAILM)], m_buft)
        remap(TAILM, rowt_d, idxt_buf)
        pltpu.sync_copy(m_buft, sp_m.at[idxt_buf], add=True)

        base_c = base_e + cid * EPT

        def cchunk(L):
            def cbody(i, carry2):
                iv = iota16 + i * 16
                e = lax.shift_right_logical(iv, 2)
                k = lax.bitwise_and(iv, 3)
                cwv = plsc.load_gather(cwb, [e])
                de = jnp.minimum(e * 3 + k, 3 * L - 1)
                dv = plsc.load_gather(diffb, [de])
                val = jnp.where(k < 3, dv * cwv, 0.0)
                srcv = plsc.load_gather(srci, [e])
                fd = srcv * 4 + k
                for q in range(4):
                    mq = lax.shift_right_logical(iota16, 2) == q
                    plsc.addupdate_scatter(cacc, [fd], val, mask=mq)
                return carry2

            lax.fori_loop(0, (4 * L) // 16, cbody, 0)

        def cbody_outer(c, carry):
            off = base_c + c * _CHI
            pltpu.sync_copy(src_hbm.at[pl.ds(off, _CHI)], srci)
            pltpu.sync_copy(cw_hbm.at[pl.ds(off, _CHI)], cwb)
            pltpu.sync_copy(diff_hbm.at[pl.ds(off * 3, 3 * _CHI)], diffb)
            cchunk(_CHI)
            return carry

        lax.fori_loop(0, NCHC, cbody_outer, 0)

        ctoff = base_c + NCHC * _CHI
        pltpu.sync_copy(src_hbm.at[pl.ds(ctoff, TAILC)],
                        srci.at[pl.ds(0, TAILC)])
        pltpu.sync_copy(cw_hbm.at[pl.ds(ctoff, TAILC)],
                        cwb.at[pl.ds(0, TAILC)])
        pltpu.sync_copy(diff_hbm.at[pl.ds(ctoff * 3, 3 * TAILC)],
                        diffb.at[pl.ds(0, 3 * TAILC)])
        cchunk(TAILC)

        plsc.subcore_barrier()

        @pl.when(sid < NS - 1)
        def _():
            pltpu.sync_copy(sp_m.at[pl.ds(sid * RPT, RPT)],
                            mp_out.at[cid, pl.ds(sid * RPT, RPT)])

        @pl.when(sid == NS - 1)
        def _():
            pltpu.sync_copy(sp_m.at[pl.ds((NS - 1) * RPT, RLAST)],
                            mp_out.at[cid, pl.ds((NS - 1) * RPT, RLAST)])

        pltpu.sync_copy(cacc, cp_out.at[wid])

    return scatter_k


def _silu(x):
    return x * jax.nn.sigmoid(x)


def _edge_body(hs_ref, hd_ref, dist_ref, temb_ref,
               ws_ref, wd_ref, wq_ref, wt_ref, be1_ref,
               we2_ref, be2_ref, wc1_ref, bc1_ref, wc2_ref,
               m_ref, cw_ref):
    f32 = jnp.float32
    x = jnp.dot(hs_ref[...], ws_ref[...], preferred_element_type=f32)
    x = x + jnp.dot(hd_ref[...], wd_ref[...], preferred_element_type=f32)
    x = x + jnp.dot(temb_ref[...], wt_ref[...], preferred_element_type=f32)
    x = x + dist_ref[...] * wq_ref[...]
    x = x + be1_ref[...]
    m1 = _silu(x)
    y = jnp.dot(m1, we2_ref[...], preferred_element_type=f32) + be2_ref[...]
    m = _silu(y)
    m_ref[...] = m
    z = jnp.dot(m, wc1_ref[...], preferred_element_type=f32) + bc1_ref[...]
    t2 = _silu(z)
    cw_ref[...] = jnp.sum(t2 * wc2_ref[...], axis=1, keepdims=True)


def _edge_mlp(hs, hd, dist_sq, t_emb, W_e1, b_e1, W_e2, b_e2,
              W_c1, b_c1, W_c2):
    E, D = hs.shape
    T = t_emb.shape[1]
    BLK = 512
    assert E % BLK == 0
    ws = W_e1[:D]
    wd = W_e1[D:2 * D]
    wq = W_e1[2 * D:2 * D + 1]
    wt = W_e1[2 * D + 1:]
    grid = (E // BLK,)
    full = lambda shape: pl.BlockSpec(shape, lambda i: (0, 0))
    blk = lambda w: pl.BlockSpec((BLK, w), lambda i: (i, 0))
    return pl.pallas_call(
        _edge_body,
        grid=grid,
        in_specs=[
            blk(D), blk(D), blk(1), blk(T),
            full((D, D)), full((D, D)), full((1, D)), full((T, D)),
            full((1, D)), full((D, D)), full((1, D)),
            full((D, D)), full((1, D)), full((1, D)),
        ],
        out_specs=[blk(D), blk(1)],
        out_shape=[jax.ShapeDtypeStruct((E, D), jnp.float32),
                   jax.ShapeDtypeStruct((E, 1), jnp.float32)],
    )(hs, hd, dist_sq, t_emb,
      ws, wd, wq, wt, b_e1.reshape(1, D), W_e2, b_e2.reshape(1, D),
      W_c1, b_c1.reshape(1, D), W_c2.reshape(1, D))


def _node_body(h_ref, mp_ref, t_ref, wh_ref, wm_ref,
               wt_ref, bn1_ref, wn2_ref, bn2_ref, out_ref):
    f32 = jnp.float32
    h = h_ref[...]
    mi = mp_ref[0]
    x = jnp.dot(h, wh_ref[...], preferred_element_type=f32)
    x = x + jnp.dot(mi, wm_ref[...], preferred_element_type=f32)
    x = x + jnp.dot(t_ref[...], wt_ref[...], preferred_element_type=f32)
    x = x + bn1_ref[...]
    u = _silu(x)
    out_ref[...] = h + jnp.dot(u, wn2_ref[...], preferred_element_type=f32) \
        + bn2_ref[...]


def _node_mlp(h, mp, t_emb_nodes, W_n1, b_n1, W_n2, b_n2):
    N, D = h.shape
    T = t_emb_nodes.shape[1]
    NC = mp.shape[0]
    HN = N // NC
    BLK = 1000
    assert N % BLK == 0 and HN % BLK == 0
    NBC = HN // BLK
    wh = W_n1[:D]
    wm = W_n1[D:2 * D]
    wt = W_n1[2 * D:]
    grid = (N // BLK,)
    full = lambda shape: pl.BlockSpec(shape, lambda i: (0, 0))
    return pl.pallas_call(
        _node_body,
        grid=grid,
        in_specs=[
            pl.BlockSpec((BLK, D), lambda i: (i, 0)),
            pl.BlockSpec((1, BLK, D), lambda i: (i // NBC, i % NBC, 0)),
            pl.BlockSpec((BLK, T), lambda i: (i, 0)),
            full((D, D)), full((D, D)), full((T, D)),
            full((1, D)), full((D, D)), full((1, D)),
        ],
        out_specs=pl.BlockSpec((BLK, D), lambda i: (i, 0)),
        out_shape=jax.ShapeDtypeStruct((N, D), jnp.float32),
    )(h, mp, t_emb_nodes, wh, wm, wt,
      b_n1.reshape(1, D), W_n2, b_n2.reshape(1, D))


def _coord_sum_body(cp_ref, out_ref):
    out_ref[...] = jnp.sum(cp_ref[...], axis=0, keepdims=True)


def _coord_sum(cp):
    _, L = cp.shape
    return pl.pallas_call(
        _coord_sum_body,
        out_shape=jax.ShapeDtypeStruct((1, L), jnp.float32),
    )(cp)


def kernel(h, diff_cart, dist_sq, edge_src, edge_dst, t_emb_edges,
           t_emb_nodes, W_e1, b_e1, W_e2, b_e2, W_c1, b_c1, W_c2,
           W_n1, b_n1, W_n2, b_n2):
    N, D = h.shape
    E = edge_src.shape[0]
    info = plsc.get_sparse_core_info()
    NC, NS = info.num_cores, info.num_subcores
    NW = NC * NS
    HN = N // NC
    HACC = -(-(HN + 1) // 8) * 8
    NCHG, TAILG = _chunking(E // NW)
    NCHM, TAILM = _chunking(E // NS)

    gather_k = _make_sc_gather(N, D, E)
    scatter_k = _make_sc_scatter(N, D, E)

    src2, srct = _stage_idx(edge_src, NW, NCHG, TAILG)
    dst2, dstt = _stage_idx(edge_dst, NW, NCHG, TAILG)
    dstm2, dstmt = _stage_idx(edge_dst, NS, NCHM, TAILM)

    hs, hd = gather_k(h, src2, srct, dst2, dstt)
    m, cw = _edge_mlp(hs, hd, dist_sq, t_emb_edges,
                      W_e1, b_e1, W_e2, b_e2, W_c1, b_c1, W_c2)

    zm = jnp.zeros((HACC, D), jnp.float32)
    zc = jnp.zeros((N * 4,), jnp.float32)
    mp, cp = scatter_k(m, dstm2, dstmt, edge_src, cw.reshape(E),
                       diff_cart.reshape(3 * E), zm, zc)

    h_update = _node_mlp(h, mp, t_emb_nodes, W_n1, b_n1, W_n2, b_n2)
    coord_update = _coord_sum(cp).reshape(N, 4)[:, :3]
    return (h_update, coord_update)
